# baseline (speedup 1.0000x reference)
import functools

import jax
import jax.numpy as jnp
from jax import lax
from jax.experimental import pallas as pl
from jax.experimental.pallas import tpu as pltpu
from jax.experimental.pallas import tpu_sc as plsc

HIDDEN_DIM = 64
BATCH = 16384

_NUM_CORES = 2
_NUM_SUBCORES = 16
_NUM_WORKERS = _NUM_CORES * _NUM_SUBCORES
_B_PER_W = BATCH // _NUM_WORKERS
_VREGS = _B_PER_W // 16

_mesh = plsc.VectorSubcoreMesh(core_axis_name="c", subcore_axis_name="s")


@functools.partial(
    pl.kernel,
    mesh=_mesh,
    out_type=jax.ShapeDtypeStruct((BATCH, HIDDEN_DIM), jnp.float32),
    scratch_types=[
        pltpu.VMEM((_B_PER_W,), jnp.int32),
        pltpu.SemaphoreType.DMA,
    ],
    compiler_params=pltpu.CompilerParams(needs_layout_passes=False),
)
def _gather_kernel(idx_hbm, table_hbm, out_hbm, idx_v, sem):
    wid = lax.axis_index("s") * _NUM_CORES + lax.axis_index("c")
    base = wid * _B_PER_W
    pltpu.sync_copy(idx_hbm.at[pl.ds(base, _B_PER_W)], idx_v)
    iota = lax.iota(jnp.int32, 16)
    for g in range(_VREGS):
        vec = idx_v[pl.ds(g * 16, 16)]
        for l in range(16):
            sj = lax.reduce_sum(jnp.where(iota == l, vec, 0), axes=(0,))
            pltpu.async_copy(
                table_hbm.at[sj], out_hbm.at[base + g * 16 + l], sem
            )
    pltpu.make_async_copy(
        table_hbm.at[pl.ds(0, _B_PER_W)],
        out_hbm.at[pl.ds(base, _B_PER_W)],
        sem,
    ).wait()


def kernel(x, in_embed):
    return _gather_kernel(x.astype(jnp.int32), in_embed)


# speedup vs baseline: 1.6630x; 1.6630x over previous
import functools

import jax
import jax.numpy as jnp
from jax import lax
from jax.experimental import pallas as pl
from jax.experimental.pallas import tpu as pltpu
from jax.experimental.pallas import tpu_sc as plsc

HIDDEN_DIM = 64
BATCH = 16384

_NUM_CORES = 2
_NUM_SUBCORES = 16
_NUM_WORKERS = _NUM_CORES * _NUM_SUBCORES
_B_PER_W = BATCH // _NUM_WORKERS
_VREGS = _B_PER_W // 16

_mesh = plsc.VectorSubcoreMesh(core_axis_name="c", subcore_axis_name="s")


@functools.partial(
    pl.kernel,
    mesh=_mesh,
    out_type=jax.ShapeDtypeStruct((BATCH, HIDDEN_DIM), jnp.float32),
    scratch_types=[
        pltpu.VMEM((_B_PER_W,), jnp.int32),
        pltpu.VMEM((_B_PER_W, HIDDEN_DIM), jnp.float32),
        pltpu.SemaphoreType.DMA,
    ],
    compiler_params=pltpu.CompilerParams(needs_layout_passes=False),
)
def _gather_kernel(idx_hbm, table_hbm, out_hbm, idx_v, rows_v, sem):
    wid = lax.axis_index("s") * _NUM_CORES + lax.axis_index("c")
    base = wid * _B_PER_W
    pltpu.sync_copy(idx_hbm.at[pl.ds(base, _B_PER_W)], idx_v)
    iota = lax.iota(jnp.int32, 16)
    for g in range(_VREGS):
        vec = idx_v[pl.ds(g * 16, 16)]
        for l in range(16):
            sj = lax.reduce_sum(jnp.where(iota == l, vec, 0), axes=(0,))
            pltpu.async_copy(
                table_hbm.at[sj], rows_v.at[g * 16 + l], sem
            )
    pltpu.make_async_copy(
        table_hbm.at[pl.ds(0, _B_PER_W)],
        rows_v,
        sem,
    ).wait()
    pltpu.sync_copy(rows_v, out_hbm.at[pl.ds(base, _B_PER_W)])


def kernel(x, in_embed):
    return _gather_kernel(x.astype(jnp.int32), in_embed)
